# baseline (speedup 1.0000x reference)
import dataclasses

import jax
import jax.numpy as jnp
from jax import lax
from jax.experimental import pallas as pl
from jax.experimental.pallas import tpu as pltpu
from jax.experimental.pallas import tpu_sc as plsc

N = 10000
E = 320000
D = 128
G = 128
C = 10

NUM_CORES = 2
NUM_SUBCORES = 16
NW = NUM_CORES * NUM_SUBCORES
CHUNK = 128
TAIL = 16
N_PAD = 10240
ROWS_PER_SUB = N_PAD // NUM_SUBCORES
BLOCKS_PER_SUB = ROWS_PER_SUB // CHUNK
EPW = E // NW
NFULL = EPW // CHUNK
NQ = (NFULL - 2) // 4
LANES = 16


def _sc_body(src_hbm, dst_hbm, feat_hbm, zf_hbm, zd_hbm,
             agg_out, deg_out,
             isrcA, isrcB, idstA, idstB,
             isrct, idstt, rows0, rows1, rowst, deg_v, agg_sh,
             isemA, isemB, gsem0, gsem1, ssem0, ssem1):
    c = lax.axis_index("c")
    s = lax.axis_index("s")
    wid = c * NUM_SUBCORES + s
    r0 = s * ROWS_PER_SUB
    out0 = c * N_PAD + r0
    ebase = wid * EPW

    pltpu.sync_copy(zf_hbm, rows0)
    pltpu.sync_copy(zd_hbm, deg_v)

    @pl.loop(0, BLOCKS_PER_SUB)
    def _(b):
        pltpu.sync_copy(rows0, agg_sh.at[pl.ds(r0 + b * CHUNK, CHUNK)])

    plsc.subcore_barrier()

    ones16 = jnp.ones((LANES,), jnp.float32)

    def hist(idst):
        @pl.loop(0, CHUNK, step=LANES)
        def _(j):
            plsc.addupdate_scatter(deg_v, [idst[pl.ds(j, LANES)]], ones16)

    PAIR = 2 * CHUNK

    def idx_fetch(ci, isrc, idst, sem):
        base = jnp.minimum(ebase + ci * CHUNK, E - PAIR)
        pltpu.async_copy(src_hbm.at[pl.ds(base, PAIR)], isrc, sem)
        pltpu.async_copy(dst_hbm.at[pl.ds(base, PAIR)], idst, sem)

    def idx_wait(ci, isrc, idst, sem):
        base = jnp.minimum(ebase + ci * CHUNK, E - PAIR)
        pltpu.make_async_copy(src_hbm.at[pl.ds(base, PAIR)], isrc,
                              sem).wait()
        pltpu.make_async_copy(dst_hbm.at[pl.ds(base, PAIR)], idst,
                              sem).wait()

    def gather(isrc, rows, sem):
        pltpu.async_copy(feat_hbm.at[isrc], rows, sem)

    def gather_wait(isrc, rows, sem):
        pltpu.make_async_copy(feat_hbm.at[isrc], rows, sem).wait()

    def scatter(idst, rows, sem):
        pltpu.async_copy(rows, agg_sh.at[idst], sem, add=True)

    def scatter_wait(idst, rows, sem):
        pltpu.make_async_copy(rows, agg_sh.at[idst], sem).wait()

    def cslice(ref, k):
        return ref.at[pl.ds(k * CHUNK, CHUNK)]

    idx_fetch(0, isrcA, idstA, isemA)
    idx_fetch(2, isrcB, idstB, isemB)
    idx_wait(0, isrcA, idstA, isemA)
    idx_wait(2, isrcB, idstB, isemB)
    gather(cslice(isrcA, 0), rows0, gsem0)

    @pl.loop(0, NQ)
    def _(q):
        b = 4 * q
        nb = jnp.minimum(b + 4, NFULL - 2)
        gather_wait(cslice(isrcA, 0), rows0, gsem0)
        gather(cslice(isrcA, 1), rows1, gsem1)
        scatter(cslice(idstA, 0), rows0, ssem0)
        hist(cslice(idstA, 0))
        scatter_wait(cslice(idstA, 0), rows0, ssem0)
        gather_wait(cslice(isrcA, 1), rows1, gsem1)
        gather(cslice(isrcB, 0), rows0, gsem0)
        scatter(cslice(idstA, 1), rows1, ssem1)
        hist(cslice(idstA, 1))
        scatter_wait(cslice(idstA, 1), rows1, ssem1)
        idx_fetch(nb + 0, isrcA, idstA, isemA)
        gather_wait(cslice(isrcB, 0), rows0, gsem0)
        gather(cslice(isrcB, 1), rows1, gsem1)
        scatter(cslice(idstB, 0), rows0, ssem0)
        hist(cslice(idstB, 0))
        scatter_wait(cslice(idstB, 0), rows0, ssem0)
        gather_wait(cslice(isrcB, 1), rows1, gsem1)
        scatter(cslice(idstB, 1), rows1, ssem1)
        hist(cslice(idstB, 1))
        scatter_wait(cslice(idstB, 1), rows1, ssem1)
        idx_fetch(nb + 2, isrcB, idstB, isemB)
        idx_wait(nb + 0, isrcA, idstA, isemA)
        idx_wait(nb + 2, isrcB, idstB, isemB)
        gather(cslice(isrcA, 0), rows0, gsem0)

    tbase = ebase + NFULL * CHUNK
    pltpu.sync_copy(src_hbm.at[pl.ds(tbase, TAIL)], isrct)
    pltpu.sync_copy(dst_hbm.at[pl.ds(tbase, TAIL)], idstt)
    gather_wait(cslice(isrcA, 0), rows0, gsem0)
    gather(cslice(isrcA, 1), rows1, gsem1)
    pltpu.async_copy(feat_hbm.at[isrct], rowst, isemA)
    scatter(cslice(idstA, 0), rows0, ssem0)
    hist(cslice(idstA, 0))
    scatter_wait(cslice(idstA, 0), rows0, ssem0)
    gather_wait(cslice(isrcA, 1), rows1, gsem1)
    scatter(cslice(idstA, 1), rows1, ssem1)
    hist(cslice(idstA, 1))
    scatter_wait(cslice(idstA, 1), rows1, ssem1)
    pltpu.make_async_copy(feat_hbm.at[isrct], rowst, isemA).wait()
    pltpu.sync_copy(rowst, agg_sh.at[idstt], add=True)
    plsc.addupdate_scatter(deg_v, [idstt[...]], ones16)

    plsc.subcore_barrier()

    @pl.loop(0, BLOCKS_PER_SUB)
    def _(b):
        pltpu.sync_copy(agg_sh.at[pl.ds(r0 + b * CHUNK, CHUNK)], rows0)
        pltpu.sync_copy(rows0, agg_out.at[pl.ds(out0 + b * CHUNK, CHUNK)])

    pltpu.sync_copy(deg_v, deg_out.at[pl.ds(wid * N_PAD, N_PAD)])


def _sc_segment_sum(src2, dst2, features, zeros_feat, zeros_deg):
    mesh = plsc.VectorSubcoreMesh(core_axis_name="c", subcore_axis_name="s")
    cp = pltpu.CompilerParams()
    if "needs_layout_passes" in pltpu.CompilerParams.__dataclass_fields__:
        cp = dataclasses.replace(cp, needs_layout_passes=False)
    kern = pl.kernel(
        _sc_body,
        compiler_params=cp,
        out_type=(
            jax.ShapeDtypeStruct((NUM_CORES * N_PAD, D), jnp.float32),
            jax.ShapeDtypeStruct((NW * N_PAD,), jnp.float32),
        ),
        mesh=mesh,
        scratch_types=[
            pltpu.VMEM((2 * CHUNK,), jnp.int32),
            pltpu.VMEM((2 * CHUNK,), jnp.int32),
            pltpu.VMEM((2 * CHUNK,), jnp.int32),
            pltpu.VMEM((2 * CHUNK,), jnp.int32),
            pltpu.VMEM((TAIL,), jnp.int32),
            pltpu.VMEM((TAIL,), jnp.int32),
            pltpu.VMEM((CHUNK, D), jnp.float32),
            pltpu.VMEM((CHUNK, D), jnp.float32),
            pltpu.VMEM((TAIL, D), jnp.float32),
            pltpu.VMEM((N_PAD,), jnp.float32),
            pltpu.VMEM_SHARED((N_PAD, D), jnp.float32),
            pltpu.SemaphoreType.DMA,
            pltpu.SemaphoreType.DMA,
            pltpu.SemaphoreType.DMA,
            pltpu.SemaphoreType.DMA,
            pltpu.SemaphoreType.DMA,
            pltpu.SemaphoreType.DMA,
        ],
    )
    return kern(src2, dst2, features, zeros_feat, zeros_deg)


def _tc_body(feat, agg2, degw, batch_row, ws, wn, wc, bc,
             scores_out, gemb_out):
    agg = agg2[0:N, :] + agg2[N_PAD:N_PAD + N, :]
    ones_w = jnp.ones((NW, 1), jnp.float32)
    deg = lax.dot_general(degw[:, 0:N], ones_w, (((0,), (0,)), ((), ())),
                          preferred_element_type=jnp.float32)
    mean = agg / jnp.maximum(deg, 1.0)
    h = jnp.dot(feat[...], ws[...], preferred_element_type=jnp.float32)
    h = h + jnp.dot(mean, wn[...], preferred_element_type=jnp.float32)
    emb = jnp.maximum(h, 0.0)
    iota_g = lax.broadcasted_iota(jnp.int32, (G, N), 0)
    onehot_t = (batch_row[...] == iota_g).astype(jnp.float32)
    gemb = jnp.dot(onehot_t, emb, preferred_element_type=jnp.float32)
    gemb_out[...] = gemb
    scores_out[...] = jnp.dot(gemb, wc[...],
                              preferred_element_type=jnp.float32) + bc[...]


def kernel(features, edge_index, batch, W_self, W_neigh, W_cls, b_cls):
    src2 = edge_index[0].astype(jnp.int32)
    dst2 = edge_index[1].astype(jnp.int32)
    zeros_feat = jnp.zeros((CHUNK, D), jnp.float32)
    zeros_deg = jnp.zeros((N_PAD,), jnp.float32)

    agg2, degw = _sc_segment_sum(src2, dst2, features, zeros_feat, zeros_deg)

    batch_row = batch.astype(jnp.int32).reshape(1, N)
    scores, gemb = pl.pallas_call(
        _tc_body,
        out_shape=(
            jax.ShapeDtypeStruct((G, C), jnp.float32),
            jax.ShapeDtypeStruct((G, D), jnp.float32),
        ),
    )(features, agg2, degw.reshape(NW, N_PAD), batch_row, W_self, W_neigh,
      W_cls, b_cls.reshape(1, C))
    return (scores, gemb)


# speedup vs baseline: 1.0612x; 1.0612x over previous
import dataclasses

import jax
import jax.numpy as jnp
from jax import lax
from jax.experimental import pallas as pl
from jax.experimental.pallas import tpu as pltpu
from jax.experimental.pallas import tpu_sc as plsc

N = 10000
E = 320000
D = 128
G = 128
C = 10

NUM_CORES = 2
NUM_SUBCORES = 16
NW = NUM_CORES * NUM_SUBCORES
CHUNK = 128
TAIL = 16
N_PAD = 10240
ROWS_PER_SUB = N_PAD // NUM_SUBCORES
BLOCKS_PER_SUB = ROWS_PER_SUB // CHUNK
EPW = E // NW
NFULL = EPW // CHUNK
NQ = (NFULL - 2) // 4
LANES = 16


def _sc_body(ei_hbm, feat_hbm, zf_hbm, zd_hbm,
             agg_out, deg_out,
             isrc0, isrc1, isrc2, isrc3, idst0, idst1, idst2, idst3,
             isrct, idstt, rows0, rows1, rowst, deg_v, agg_sh,
             isem, gsem0, gsem1, ssem0, ssem1):
    c = lax.axis_index("c")
    s = lax.axis_index("s")
    wid = c * NUM_SUBCORES + s
    r0 = s * ROWS_PER_SUB
    out0 = c * N_PAD + r0
    ebase = wid * EPW

    pltpu.sync_copy(zf_hbm, rows0)
    pltpu.sync_copy(zd_hbm, deg_v)

    @pl.loop(0, BLOCKS_PER_SUB)
    def _(b):
        pltpu.sync_copy(rows0, agg_sh.at[pl.ds(r0 + b * CHUNK, CHUNK)])

    plsc.subcore_barrier()

    ones16 = jnp.ones((LANES,), jnp.float32)

    def hist(idst):
        @pl.loop(0, CHUNK, step=LANES)
        def _(j):
            plsc.addupdate_scatter(deg_v, [idst[pl.ds(j, LANES)]], ones16)

    def idx_fetch(ci, isrc, idst):
        base = jnp.minimum(ebase + ci * CHUNK, E - CHUNK)
        pltpu.async_copy(ei_hbm.at[pl.ds(base, CHUNK)], isrc, isem)
        pltpu.async_copy(ei_hbm.at[pl.ds(E + base, CHUNK)], idst, isem)

    def idx_wait(ci, isrc, idst):
        base = jnp.minimum(ebase + ci * CHUNK, E - CHUNK)
        pltpu.make_async_copy(ei_hbm.at[pl.ds(base, CHUNK)], isrc,
                              isem).wait()
        pltpu.make_async_copy(ei_hbm.at[pl.ds(E + base, CHUNK)], idst,
                              isem).wait()

    def gather(isrc, rows, sem):
        pltpu.async_copy(feat_hbm.at[isrc], rows, sem)

    def gather_wait(isrc, rows, sem):
        pltpu.make_async_copy(feat_hbm.at[isrc], rows, sem).wait()

    def scatter(idst, rows, sem):
        pltpu.async_copy(rows, agg_sh.at[idst], sem, add=True)

    def scatter_wait(idst, rows, sem):
        pltpu.make_async_copy(rows, agg_sh.at[idst], sem).wait()

    idx_fetch(0, isrc0, idst0)
    idx_fetch(1, isrc1, idst1)
    idx_fetch(2, isrc2, idst2)
    idx_fetch(3, isrc3, idst3)
    idx_wait(0, isrc0, idst0)
    idx_wait(1, isrc1, idst1)
    idx_wait(2, isrc2, idst2)
    idx_wait(3, isrc3, idst3)
    gather(isrc0, rows0, gsem0)

    @pl.loop(0, NQ)
    def _(q):
        b = 4 * q
        nb = jnp.minimum(b + 4, NFULL - 2)
        gather_wait(isrc0, rows0, gsem0)
        gather(isrc1, rows1, gsem1)
        scatter(idst0, rows0, ssem0)
        hist(idst0)
        scatter_wait(idst0, rows0, ssem0)
        gather_wait(isrc1, rows1, gsem1)
        gather(isrc2, rows0, gsem0)
        scatter(idst1, rows1, ssem1)
        hist(idst1)
        scatter_wait(idst1, rows1, ssem1)
        idx_fetch(nb + 0, isrc0, idst0)
        idx_fetch(nb + 1, isrc1, idst1)
        gather_wait(isrc2, rows0, gsem0)
        gather(isrc3, rows1, gsem1)
        scatter(idst2, rows0, ssem0)
        hist(idst2)
        scatter_wait(idst2, rows0, ssem0)
        gather_wait(isrc3, rows1, gsem1)
        idx_fetch(nb + 2, isrc2, idst2)
        scatter(idst3, rows1, ssem1)
        hist(idst3)
        scatter_wait(idst3, rows1, ssem1)
        idx_fetch(nb + 3, isrc3, idst3)
        idx_wait(nb + 0, isrc0, idst0)
        idx_wait(nb + 1, isrc1, idst1)
        idx_wait(nb + 2, isrc2, idst2)
        idx_wait(nb + 3, isrc3, idst3)
        gather(isrc0, rows0, gsem0)

    tbase = ebase + NFULL * CHUNK
    pltpu.sync_copy(ei_hbm.at[pl.ds(tbase, TAIL)], isrct)
    pltpu.sync_copy(ei_hbm.at[pl.ds(E + tbase, TAIL)], idstt)
    gather_wait(isrc0, rows0, gsem0)
    gather(isrc1, rows1, gsem1)
    pltpu.async_copy(feat_hbm.at[isrct], rowst, isem)
    scatter(idst0, rows0, ssem0)
    hist(idst0)
    scatter_wait(idst0, rows0, ssem0)
    gather_wait(isrc1, rows1, gsem1)
    scatter(idst1, rows1, ssem1)
    hist(idst1)
    scatter_wait(idst1, rows1, ssem1)
    pltpu.make_async_copy(feat_hbm.at[isrct], rowst, isem).wait()
    pltpu.sync_copy(rowst, agg_sh.at[idstt], add=True)
    plsc.addupdate_scatter(deg_v, [idstt[...]], ones16)

    plsc.subcore_barrier()

    @pl.loop(0, BLOCKS_PER_SUB)
    def _(b):
        pltpu.sync_copy(agg_sh.at[pl.ds(r0 + b * CHUNK, CHUNK)], rows0)
        pltpu.sync_copy(rows0, agg_out.at[pl.ds(out0 + b * CHUNK, CHUNK)])

    pltpu.sync_copy(deg_v, deg_out.at[pl.ds(wid * N_PAD, N_PAD)])


def _sc_segment_sum(ei_flat, features, zeros_feat, zeros_deg):
    mesh = plsc.VectorSubcoreMesh(core_axis_name="c", subcore_axis_name="s")
    cp = pltpu.CompilerParams()
    if "needs_layout_passes" in pltpu.CompilerParams.__dataclass_fields__:
        cp = dataclasses.replace(cp, needs_layout_passes=False)
    kern = pl.kernel(
        _sc_body,
        compiler_params=cp,
        out_type=(
            jax.ShapeDtypeStruct((NUM_CORES * N_PAD, D), jnp.float32),
            jax.ShapeDtypeStruct((NW * N_PAD,), jnp.float32),
        ),
        mesh=mesh,
        scratch_types=[
            pltpu.VMEM((CHUNK,), jnp.int32),
            pltpu.VMEM((CHUNK,), jnp.int32),
            pltpu.VMEM((CHUNK,), jnp.int32),
            pltpu.VMEM((CHUNK,), jnp.int32),
            pltpu.VMEM((CHUNK,), jnp.int32),
            pltpu.VMEM((CHUNK,), jnp.int32),
            pltpu.VMEM((CHUNK,), jnp.int32),
            pltpu.VMEM((CHUNK,), jnp.int32),
            pltpu.VMEM((TAIL,), jnp.int32),
            pltpu.VMEM((TAIL,), jnp.int32),
            pltpu.VMEM((CHUNK, D), jnp.float32),
            pltpu.VMEM((CHUNK, D), jnp.float32),
            pltpu.VMEM((TAIL, D), jnp.float32),
            pltpu.VMEM((N_PAD,), jnp.float32),
            pltpu.VMEM_SHARED((N_PAD, D), jnp.float32),
            pltpu.SemaphoreType.DMA,
            pltpu.SemaphoreType.DMA,
            pltpu.SemaphoreType.DMA,
            pltpu.SemaphoreType.DMA,
            pltpu.SemaphoreType.DMA,
        ],
    )
    return kern(ei_flat, features, zeros_feat, zeros_deg)


def _tc_body(feat, agg2, degw, batch_row, ws, wn, wc, bc,
             scores_out, gemb_out):
    agg = agg2[0:N, :] + agg2[N_PAD:N_PAD + N, :]
    ones_w = jnp.ones((NW, 1), jnp.float32)
    deg = lax.dot_general(degw[:, 0:N], ones_w, (((0,), (0,)), ((), ())),
                          preferred_element_type=jnp.float32)
    mean = agg / jnp.maximum(deg, 1.0)
    h = jnp.dot(feat[...], ws[...], preferred_element_type=jnp.float32)
    h = h + jnp.dot(mean, wn[...], preferred_element_type=jnp.float32)
    emb = jnp.maximum(h, 0.0)
    iota_g = lax.broadcasted_iota(jnp.int32, (G, N), 0)
    onehot_t = (batch_row[...] == iota_g).astype(jnp.float32)
    gemb = jnp.dot(onehot_t, emb, preferred_element_type=jnp.float32)
    gemb_out[...] = gemb
    scores_out[...] = jnp.dot(gemb, wc[...],
                              preferred_element_type=jnp.float32) + bc[...]


def kernel(features, edge_index, batch, W_self, W_neigh, W_cls, b_cls):
    ei_flat = edge_index.astype(jnp.int32).reshape(2 * E)
    zeros_feat = jnp.zeros((CHUNK, D), jnp.float32)
    zeros_deg = jnp.zeros((N_PAD,), jnp.float32)

    agg2, degw = _sc_segment_sum(ei_flat, features, zeros_feat, zeros_deg)

    batch_row = batch.astype(jnp.int32).reshape(1, N)
    scores, gemb = pl.pallas_call(
        _tc_body,
        out_shape=(
            jax.ShapeDtypeStruct((G, C), jnp.float32),
            jax.ShapeDtypeStruct((G, D), jnp.float32),
        ),
    )(features, agg2, degw.reshape(NW, N_PAD), batch_row, W_self, W_neigh,
      W_cls, b_cls.reshape(1, C))
    return (scores, gemb)


# speedup vs baseline: 1.0751x; 1.0132x over previous
import dataclasses

import jax
import jax.numpy as jnp
from jax import lax
from jax.experimental import pallas as pl
from jax.experimental.pallas import tpu as pltpu
from jax.experimental.pallas import tpu_sc as plsc

N = 10000
E = 320000
D = 128
G = 128
C = 10

NUM_CORES = 2
NUM_SUBCORES = 16
NW = NUM_CORES * NUM_SUBCORES
CHUNK = 128
TAIL = 16
N_PAD = 10240
ROWS_PER_SUB = N_PAD // NUM_SUBCORES
BLOCKS_PER_SUB = ROWS_PER_SUB // CHUNK
EPW = E // NW
NFULL = EPW // CHUNK
NQ = (NFULL - 2) // 4
LANES = 16


def _sc_body(ei_hbm, feat_hbm, zf_hbm, zd_hbm,
             agg_out, deg_out,
             isrc0, isrc1, isrc2, isrc3, idst0, idst1, idst2, idst3,
             isrct, idstt, rows0, rows1, rowst, deg_v, agg_sh,
             isem, gsem0, gsem1, ssem0, ssem1):
    c = lax.axis_index("c")
    s = lax.axis_index("s")
    wid = c * NUM_SUBCORES + s
    r0 = s * ROWS_PER_SUB
    out0 = c * N_PAD + r0
    ebase = wid * EPW

    pltpu.sync_copy(zf_hbm, rows0)
    pltpu.sync_copy(zd_hbm, deg_v)

    @pl.loop(0, BLOCKS_PER_SUB)
    def _(b):
        pltpu.sync_copy(rows0, agg_sh.at[pl.ds(r0 + b * CHUNK, CHUNK)])

    plsc.subcore_barrier()

    ones16 = jnp.ones((LANES,), jnp.float32)

    def hist(idst):
        @pl.loop(0, CHUNK, step=LANES)
        def _(j):
            plsc.addupdate_scatter(deg_v, [idst[pl.ds(j, LANES)]], ones16)

    def idx_fetch(ci, isrc, idst):
        base = jnp.minimum(ebase + ci * CHUNK, E - CHUNK)
        pltpu.async_copy(ei_hbm.at[pl.ds(base, CHUNK)], isrc, isem)
        pltpu.async_copy(ei_hbm.at[pl.ds(E + base, CHUNK)], idst, isem)

    def idx_wait(ci, isrc, idst):
        base = jnp.minimum(ebase + ci * CHUNK, E - CHUNK)
        pltpu.make_async_copy(ei_hbm.at[pl.ds(base, CHUNK)], isrc,
                              isem).wait()
        pltpu.make_async_copy(ei_hbm.at[pl.ds(E + base, CHUNK)], idst,
                              isem).wait()

    def gather(isrc, rows, sem):
        pltpu.async_copy(feat_hbm.at[isrc], rows, sem)

    def gather_wait(isrc, rows, sem):
        pltpu.make_async_copy(feat_hbm.at[isrc], rows, sem).wait()

    def scatter(idst, rows, sem):
        pltpu.async_copy(rows, agg_sh.at[idst], sem, add=True)

    def scatter_wait(idst, rows, sem):
        pltpu.make_async_copy(rows, agg_sh.at[idst], sem).wait()

    idx_fetch(0, isrc0, idst0)
    idx_fetch(1, isrc1, idst1)
    idx_fetch(2, isrc2, idst2)
    idx_fetch(3, isrc3, idst3)
    idx_wait(0, isrc0, idst0)
    idx_wait(1, isrc1, idst1)
    idx_wait(2, isrc2, idst2)
    idx_wait(3, isrc3, idst3)
    gather(isrc0, rows0, gsem0)

    @pl.loop(0, NQ)
    def _(q):
        b = 4 * q
        nb = jnp.minimum(b + 4, NFULL - 2)
        gather_wait(isrc0, rows0, gsem0)
        gather(isrc1, rows1, gsem1)
        scatter(idst0, rows0, ssem0)
        hist(idst0)
        scatter_wait(idst0, rows0, ssem0)
        gather_wait(isrc1, rows1, gsem1)
        gather(isrc2, rows0, gsem0)
        scatter(idst1, rows1, ssem1)
        hist(idst1)
        scatter_wait(idst1, rows1, ssem1)
        idx_fetch(nb + 0, isrc0, idst0)
        idx_fetch(nb + 1, isrc1, idst1)
        gather_wait(isrc2, rows0, gsem0)
        gather(isrc3, rows1, gsem1)
        scatter(idst2, rows0, ssem0)
        hist(idst2)
        scatter_wait(idst2, rows0, ssem0)
        gather_wait(isrc3, rows1, gsem1)
        idx_fetch(nb + 2, isrc2, idst2)
        scatter(idst3, rows1, ssem1)
        hist(idst3)
        scatter_wait(idst3, rows1, ssem1)
        idx_fetch(nb + 3, isrc3, idst3)
        idx_wait(nb + 0, isrc0, idst0)
        idx_wait(nb + 1, isrc1, idst1)
        idx_wait(nb + 2, isrc2, idst2)
        idx_wait(nb + 3, isrc3, idst3)
        gather(isrc0, rows0, gsem0)

    tbase = ebase + NFULL * CHUNK
    pltpu.sync_copy(ei_hbm.at[pl.ds(tbase, TAIL)], isrct)
    pltpu.sync_copy(ei_hbm.at[pl.ds(E + tbase, TAIL)], idstt)
    gather_wait(isrc0, rows0, gsem0)
    gather(isrc1, rows1, gsem1)
    pltpu.async_copy(feat_hbm.at[isrct], rowst, isem)
    scatter(idst0, rows0, ssem0)
    hist(idst0)
    scatter_wait(idst0, rows0, ssem0)
    gather_wait(isrc1, rows1, gsem1)
    scatter(idst1, rows1, ssem1)
    hist(idst1)
    scatter_wait(idst1, rows1, ssem1)
    pltpu.make_async_copy(feat_hbm.at[isrct], rowst, isem).wait()
    pltpu.sync_copy(rowst, agg_sh.at[idstt], add=True)
    plsc.addupdate_scatter(deg_v, [idstt[...]], ones16)

    plsc.subcore_barrier()

    @pl.loop(0, BLOCKS_PER_SUB)
    def _(b):
        pltpu.sync_copy(agg_sh.at[pl.ds(r0 + b * CHUNK, CHUNK)], rows0)
        pltpu.sync_copy(rows0, agg_out.at[pl.ds(out0 + b * CHUNK, CHUNK)])

    pltpu.sync_copy(deg_v, deg_out.at[pl.ds(wid * N_PAD, N_PAD)])


def _sc_segment_sum(ei_flat, features, zeros_feat, zeros_deg):
    mesh = plsc.VectorSubcoreMesh(core_axis_name="c", subcore_axis_name="s")
    cp = pltpu.CompilerParams()
    if "needs_layout_passes" in pltpu.CompilerParams.__dataclass_fields__:
        cp = dataclasses.replace(cp, needs_layout_passes=False)
    kern = pl.kernel(
        _sc_body,
        compiler_params=cp,
        out_type=(
            jax.ShapeDtypeStruct((NUM_CORES * N_PAD, D), jnp.float32),
            jax.ShapeDtypeStruct((NW * N_PAD,), jnp.float32),
        ),
        mesh=mesh,
        scratch_types=[
            pltpu.VMEM((CHUNK,), jnp.int32),
            pltpu.VMEM((CHUNK,), jnp.int32),
            pltpu.VMEM((CHUNK,), jnp.int32),
            pltpu.VMEM((CHUNK,), jnp.int32),
            pltpu.VMEM((CHUNK,), jnp.int32),
            pltpu.VMEM((CHUNK,), jnp.int32),
            pltpu.VMEM((CHUNK,), jnp.int32),
            pltpu.VMEM((CHUNK,), jnp.int32),
            pltpu.VMEM((TAIL,), jnp.int32),
            pltpu.VMEM((TAIL,), jnp.int32),
            pltpu.VMEM((CHUNK, D), jnp.float32),
            pltpu.VMEM((CHUNK, D), jnp.float32),
            pltpu.VMEM((TAIL, D), jnp.float32),
            pltpu.VMEM((N_PAD,), jnp.float32),
            pltpu.VMEM_SHARED((N_PAD, D), jnp.float32),
            pltpu.SemaphoreType.DMA,
            pltpu.SemaphoreType.DMA,
            pltpu.SemaphoreType.DMA,
            pltpu.SemaphoreType.DMA,
            pltpu.SemaphoreType.DMA,
        ],
    )
    return kern(ei_flat, features, zeros_feat, zeros_deg)


def _tc_body(feat, agg2, degw, batch_row, ws, wn, wc, bc,
             scores_out, gemb_out):
    agg = agg2[0:N, :] + agg2[N_PAD:N_PAD + N, :]
    ones_w = jnp.ones((NW, 1), jnp.float32)
    degw2 = degw[...].reshape(NW, N_PAD)
    deg = lax.dot_general(degw2[:, 0:N], ones_w, (((0,), (0,)), ((), ())),
                          preferred_element_type=jnp.float32)
    mean = agg / jnp.maximum(deg, 1.0)
    h = jnp.dot(feat[...], ws[...], preferred_element_type=jnp.float32)
    h = h + jnp.dot(mean, wn[...], preferred_element_type=jnp.float32)
    emb = jnp.maximum(h, 0.0)
    iota_g = lax.broadcasted_iota(jnp.int32, (G, N), 0)
    onehot_t = (batch_row[...] == iota_g).astype(jnp.float32)
    gemb = jnp.dot(onehot_t, emb, preferred_element_type=jnp.float32)
    gemb_out[...] = gemb
    scores_out[...] = jnp.dot(gemb, wc[...],
                              preferred_element_type=jnp.float32) + bc[...]


def kernel(features, edge_index, batch, W_self, W_neigh, W_cls, b_cls):
    ei_flat = edge_index.astype(jnp.int32).reshape(2 * E)
    zeros_feat = jnp.zeros((CHUNK, D), jnp.float32)
    zeros_deg = jnp.zeros((N_PAD,), jnp.float32)

    agg2, degw = _sc_segment_sum(ei_flat, features, zeros_feat, zeros_deg)

    batch_row = batch.astype(jnp.int32).reshape(1, N)
    scores, gemb = pl.pallas_call(
        _tc_body,
        out_shape=(
            jax.ShapeDtypeStruct((G, C), jnp.float32),
            jax.ShapeDtypeStruct((G, D), jnp.float32),
        ),
    )(features, agg2, degw, batch_row, W_self, W_neigh,
      W_cls, b_cls.reshape(1, C))
    return (scores, gemb)
